# speedup vs baseline: 35.3423x; 35.3423x over previous
import functools

import jax
import jax.numpy as jnp
from jax.experimental import pallas as pl
from jax.experimental.pallas import tpu as pltpu


def _fused_tagger_kernel(ids_ref, lens_ref, weT_ref, tab_ref, bc_ref,
                         whhc_ref, wihw_ref, bw_ref, whhw_ref, wt_ref,
                         bt_ref, out_ref, *, S, L, BB, C, W, H, T, A):
    N = BB * S
    C4, H4 = 4 * C, 4 * H
    f32 = jnp.float32

    tab = tab_ref[...].astype(jnp.bfloat16)
    bc = bc_ref[...]
    whhc = whhc_ref[...]
    wihw = wihw_ref[...]
    bw = bw_ref[...]
    whhw = whhw_ref[...]
    wt = wt_ref[...]
    bt = bt_ref[...]

    alpha_row = jax.lax.broadcasted_iota(jnp.int32, (A, N), 0)
    gsub_c = jax.lax.broadcasted_iota(jnp.int32, (C4, N), 0)
    gmask_c = (gsub_c >= 2 * C) & (gsub_c < 3 * C)
    gsub_w = jax.lax.broadcasted_iota(jnp.int32, (H4, BB), 0)
    gmask_w = (gsub_w >= 2 * H) & (gsub_w < 3 * H)

    lens = lens_ref[0]

    def gate_acts(gates, hid, gmask):
        sig = jax.nn.sigmoid(jnp.where(gmask, gates + gates, gates))
        acts = jnp.where(gmask, sig + sig - 1.0, sig)
        return (acts[0 * hid:1 * hid], acts[1 * hid:2 * hid],
                acts[2 * hid:3 * hid], acts[3 * hid:4 * hid])

    h_c = jnp.zeros((C, N), f32)
    c_c = jnp.zeros((C, N), f32)
    for t in range(L):
        ids_t = ids_ref[0, t:t + 1, :]
        onehot = (alpha_row == ids_t).astype(jnp.bfloat16)
        gates = (jnp.dot(tab, onehot, preferred_element_type=f32)
                 + jnp.dot(whhc, h_c, preferred_element_type=f32) + bc)
        i, f, g, o = gate_acts(gates, C, gmask_c)
        c_new = f * c_c + i * g
        h_new = o * jnp.tanh(c_new)
        keep = lens > t
        h_c = jnp.where(keep, h_new, h_c)
        c_c = jnp.where(keep, c_new, c_c)

    weh = jnp.concatenate([weT_ref[0], h_c], axis=0)
    xg = jnp.dot(wihw, weh, preferred_element_type=f32) + bw
    h_w = jnp.zeros((H, BB), f32)
    c_w = jnp.zeros((H, BB), f32)
    for s in range(S):
        gates = (xg[:, s * BB:(s + 1) * BB]
                 + jnp.dot(whhw, h_w, preferred_element_type=f32))
        i, f, g, o = gate_acts(gates, H, gmask_w)
        c_w = f * c_w + i * g
        h_w = o * jnp.tanh(c_w)
        out_ref[0, s * T:(s + 1) * T, :] = (
            jnp.dot(wt, h_w, preferred_element_type=f32) + bt)


def kernel(char_emb, word_emb, w_ih_c, w_hh_c, b_ih_c, b_hh_c,
           w_ih_w, w_hh_w, b_ih_w, b_hh_w, t_w, t_b,
           word_ids, char_ids, char_lens):
    B, S = word_ids.shape
    L = char_ids.shape[2]
    A, C = char_emb.shape
    W = word_emb.shape[1]
    H = w_hh_w.shape[1]
    T = t_w.shape[0]
    f32 = jnp.float32

    BB = 256
    while B % BB:
        BB //= 2
    nb = B // BB
    N = BB * S

    ids = (char_ids.reshape(nb, BB, S, L).transpose(0, 3, 2, 1)
           .reshape(nb, L, N))
    lens = (char_lens.reshape(nb, BB, S).transpose(0, 2, 1)
            .astype(jnp.int32).reshape(nb, 1, N))
    we = jnp.take(word_emb, word_ids, axis=0)
    weT = we.reshape(nb, BB, S, W).transpose(0, 3, 2, 1).reshape(nb, W, N)

    tab = (char_emb.astype(f32) @ w_ih_c.T.astype(f32)).T
    bc = (b_ih_c + b_hh_c).astype(f32)[:, None]
    bw = (b_ih_w + b_hh_w).astype(f32)[:, None]
    bt = t_b.astype(f32)[:, None]

    grid_kernel = functools.partial(
        _fused_tagger_kernel, S=S, L=L, BB=BB, C=C, W=W, H=H, T=T, A=A)

    flops = (2 * B * S * L * A * 4 * C
             + 2 * B * S * L * C * 4 * C
             + 2 * B * S * (W + C) * 4 * H
             + 2 * B * S * H * 4 * H
             + 2 * B * S * H * T)
    transcendentals = B * S * L * 5 * C + B * S * 5 * H
    bytes_accessed = 4 * (ids.size + lens.size + weT.size + B * S * T)

    out = pl.pallas_call(
        grid_kernel,
        out_shape=jax.ShapeDtypeStruct((nb, S * T, BB), f32),
        grid=(nb,),
        in_specs=[
            pl.BlockSpec((1, L, N), lambda b: (b, 0, 0)),
            pl.BlockSpec((1, 1, N), lambda b: (b, 0, 0)),
            pl.BlockSpec((1, W, N), lambda b: (b, 0, 0)),
            pl.BlockSpec((C * 4, A), lambda b: (0, 0)),
            pl.BlockSpec((C * 4, 1), lambda b: (0, 0)),
            pl.BlockSpec((C * 4, C), lambda b: (0, 0)),
            pl.BlockSpec((H * 4, W + C), lambda b: (0, 0)),
            pl.BlockSpec((H * 4, 1), lambda b: (0, 0)),
            pl.BlockSpec((H * 4, H), lambda b: (0, 0)),
            pl.BlockSpec((T, H), lambda b: (0, 0)),
            pl.BlockSpec((T, 1), lambda b: (0, 0)),
        ],
        out_specs=pl.BlockSpec((1, S * T, BB), lambda b: (b, 0, 0)),
        compiler_params=pltpu.CompilerParams(
            dimension_semantics=("parallel",)),
        cost_estimate=pl.CostEstimate(flops=flops,
                                      transcendentals=transcendentals,
                                      bytes_accessed=bytes_accessed),
    )(ids, lens, weT, tab, bc, w_hh_c.astype(f32),
      w_ih_w.astype(f32), bw, w_hh_w.astype(f32), t_w.astype(f32), bt)

    return out.reshape(nb, S, T, BB).transpose(0, 3, 1, 2).reshape(B, S, T)
